# baseline (speedup 1.0000x reference)
import functools

import jax
import jax.numpy as jnp
import numpy as np
from jax.experimental import pallas as pl
from jax.experimental.pallas import tpu as pltpu

_B = 4
_Q = 300
_C = 80
_K = 100
_MR = 128
_OH = 512
_OW = 512


def _resize_weights(in_size: int, out_size: int) -> np.ndarray:
    scale = out_size / in_size
    sample_f = (np.arange(out_size) + 0.5) / scale - 0.5
    x = np.abs(sample_f[:, None] - np.arange(in_size)[None, :])
    w = np.maximum(0.0, 1.0 - x)
    w = w / w.sum(axis=1, keepdims=True)
    return w.astype(np.float32)


_W_UP = _resize_weights(_MR, _OH)


def _topk_box_kernel(cls_ref, box_ref, scores_out, labels_out, qidx_out,
                     boxes_out, area_out):
    x = jax.nn.sigmoid(cls_ref[...])
    flat_i = (jax.lax.broadcasted_iota(jnp.int32, (_B, _Q, _C), 1) * _C
              + jax.lax.broadcasted_iota(jnp.int32, (_B, _Q, _C), 2))
    lanes = jax.lax.broadcasted_iota(jnp.int32, (_B, 128), 1)

    def body(k, carry):
        x, s_row, l_row, q_row = carry
        m = jnp.max(x, axis=(1, 2))
        m3 = m[:, None, None]
        idx = jnp.min(jnp.where(x == m3, flat_i, jnp.int32(2**30)),
                      axis=(1, 2))
        x = jnp.where(flat_i == idx[:, None, None], -1.0, x)
        kmask = lanes == k
        s_row = jnp.where(kmask, m[:, None], s_row)
        l_row = jnp.where(kmask, (idx % _C)[:, None], l_row)
        q_row = jnp.where(kmask, (idx // _C)[:, None], q_row)
        return x, s_row, l_row, q_row

    init = (x,
            jnp.zeros((_B, 128), jnp.float32),
            jnp.zeros((_B, 128), jnp.int32),
            jnp.zeros((_B, 128), jnp.int32))
    _, s_row, l_row, q_row = jax.lax.fori_loop(0, _K, body, init)

    scores_out[...] = s_row
    labels_out[...] = l_row
    qidx_out[...] = q_row

    q_iota = jax.lax.broadcasted_iota(jnp.int32, (_B, _Q, 128), 1)
    sel = (q_iota == q_row[:, None, :]).astype(jnp.float32)
    coords = [jnp.sum(sel * box_ref[:, :, j][:, :, None], axis=1)
              for j in range(4)]
    cx, cy, w, h = coords
    x0 = jnp.clip((cx - 0.5 * w) * _OW, 0.0, _OW - 1.0)
    y0 = jnp.clip((cy - 0.5 * h) * _OH, 0.0, _OH - 1.0)
    x1 = jnp.clip((cx + 0.5 * w) * _OW, 0.0, _OW - 1.0)
    y1 = jnp.clip((cy + 0.5 * h) * _OH, 0.0, _OH - 1.0)
    boxes_out[...] = jnp.stack([x0, y0, x1, y1], axis=1)
    area_out[...] = (x1 - x0) * (y1 - y0)


def _mask_kernel(qref, sref, aref, mask_ref, wr_ref, wc_ref,
                 masks_out, score_out, keep_out):
    i = pl.program_id(0)
    m = mask_ref[0, 0]
    pos = (m > 0.0).astype(jnp.float32)
    ratio = jnp.sum(jax.nn.sigmoid(m) * pos) / (jnp.sum(pos) + 1e-6)
    tmp = jax.lax.dot_general(
        wr_ref[...], m, (((1,), (0,)), ((), ())),
        precision=jax.lax.Precision.HIGHEST,
        preferred_element_type=jnp.float32)
    up = jax.lax.dot_general(
        tmp, wc_ref[...], (((1,), (1,)), ((), ())),
        precision=jax.lax.Precision.HIGHEST,
        preferred_element_type=jnp.float32)
    bm = up > 0.0
    masks_out[0] = bm
    msum = jnp.sum(bm.astype(jnp.float32))
    sc = sref[i] * ratio
    kp = (msum > 5.0) & (aref[i] > 10.0) & (sc > 0.05)
    score_out[0, 0] = jnp.full((128,), sc, jnp.float32)
    keep_out[0, 0] = jnp.full((128,), kp, jnp.bool_)


@functools.partial(jax.jit, static_argnames=())
def kernel(class_queries_logits, masks_queries_logits, pred_boxes):
    scores_r, labels_r, qidx_r, boxes_cm, area_r = pl.pallas_call(
        _topk_box_kernel,
        out_shape=(
            jax.ShapeDtypeStruct((_B, 128), jnp.float32),
            jax.ShapeDtypeStruct((_B, 128), jnp.int32),
            jax.ShapeDtypeStruct((_B, 128), jnp.int32),
            jax.ShapeDtypeStruct((_B, 4, 128), jnp.float32),
            jax.ShapeDtypeStruct((_B, 128), jnp.float32),
        ),
    )(class_queries_logits, pred_boxes)

    q_flat = qidx_r[:, :_K].reshape(-1)
    s_flat = scores_r[:, :_K].reshape(-1)
    a_flat = area_r[:, :_K].reshape(-1)
    w_up = jnp.asarray(_W_UP)

    grid_spec = pltpu.PrefetchScalarGridSpec(
        num_scalar_prefetch=3,
        grid=(_B * _K,),
        in_specs=[
            pl.BlockSpec((1, 1, _MR, _MR),
                         lambda i, q, s, a: (i // _K, q[i], 0, 0)),
            pl.BlockSpec((_OH, _MR), lambda i, q, s, a: (0, 0)),
            pl.BlockSpec((_OW, _MR), lambda i, q, s, a: (0, 0)),
        ],
        out_specs=[
            pl.BlockSpec((1, _OH, _OW), lambda i, q, s, a: (i, 0, 0)),
            pl.BlockSpec((1, 1, 128), lambda i, q, s, a: (i, 0, 0)),
            pl.BlockSpec((1, 1, 128), lambda i, q, s, a: (i, 0, 0)),
        ],
    )
    masks_o, score_o, keep_o = pl.pallas_call(
        _mask_kernel,
        grid_spec=grid_spec,
        out_shape=(
            jax.ShapeDtypeStruct((_B * _K, _OH, _OW), jnp.bool_),
            jax.ShapeDtypeStruct((_B * _K, 1, 128), jnp.float32),
            jax.ShapeDtypeStruct((_B * _K, 1, 128), jnp.bool_),
        ),
        compiler_params=pltpu.CompilerParams(
            dimension_semantics=("arbitrary",)),
    )(q_flat, s_flat, a_flat, masks_queries_logits, w_up, w_up)

    pred_scores = score_o[:, 0, 0].reshape(_B, _K)
    keep = keep_o[:, 0, 0].reshape(_B, _K)
    masks = masks_o.reshape(_B, _K, _OH, _OW)
    boxes = boxes_cm.transpose(0, 2, 1)[:, :_K, :]
    labels = labels_r[:, :_K]
    return (pred_scores, boxes, labels, masks, keep)


# speedup vs baseline: 1.5295x; 1.5295x over previous
import functools

import jax
import jax.numpy as jnp
import numpy as np
from jax.experimental import pallas as pl
from jax.experimental.pallas import tpu as pltpu

_B = 4
_Q = 300
_C = 80
_K = 100
_MR = 128
_OH = 512
_OW = 512


def _resize_weights(in_size: int, out_size: int) -> np.ndarray:
    scale = out_size / in_size
    sample_f = (np.arange(out_size) + 0.5) / scale - 0.5
    x = np.abs(sample_f[:, None] - np.arange(in_size)[None, :])
    w = np.maximum(0.0, 1.0 - x)
    w = w / w.sum(axis=1, keepdims=True)
    return w.astype(np.float32)


_W_UP = _resize_weights(_MR, _OH)


def _topk_box_kernel(cls_ref, box_ref, scores_out, labels_out, qidx_out,
                     boxes_out, area_out):
    x = jax.nn.sigmoid(cls_ref[...])
    flat_i = (jax.lax.broadcasted_iota(jnp.int32, (_B, _Q, _C), 1) * _C
              + jax.lax.broadcasted_iota(jnp.int32, (_B, _Q, _C), 2))
    lanes = jax.lax.broadcasted_iota(jnp.int32, (_B, 128), 1)

    def body(k, carry):
        x, s_row, l_row, q_row = carry
        m = jnp.max(x, axis=(1, 2))
        m3 = m[:, None, None]
        idx = jnp.min(jnp.where(x == m3, flat_i, jnp.int32(2**30)),
                      axis=(1, 2))
        x = jnp.where(flat_i == idx[:, None, None], -1.0, x)
        kmask = lanes == k
        s_row = jnp.where(kmask, m[:, None], s_row)
        l_row = jnp.where(kmask, (idx % _C)[:, None], l_row)
        q_row = jnp.where(kmask, (idx // _C)[:, None], q_row)
        return x, s_row, l_row, q_row

    init = (x,
            jnp.zeros((_B, 128), jnp.float32),
            jnp.zeros((_B, 128), jnp.int32),
            jnp.zeros((_B, 128), jnp.int32))
    _, s_row, l_row, q_row = jax.lax.fori_loop(0, _K, body, init)

    scores_out[...] = s_row
    labels_out[...] = l_row
    qidx_out[...] = q_row

    q_iota = jax.lax.broadcasted_iota(jnp.int32, (_B, _Q, 128), 1)
    sel = (q_iota == q_row[:, None, :]).astype(jnp.float32)
    coords = [jnp.sum(sel * box_ref[:, :, j][:, :, None], axis=1)
              for j in range(4)]
    cx, cy, w, h = coords
    x0 = jnp.clip((cx - 0.5 * w) * _OW, 0.0, _OW - 1.0)
    y0 = jnp.clip((cy - 0.5 * h) * _OH, 0.0, _OH - 1.0)
    x1 = jnp.clip((cx + 0.5 * w) * _OW, 0.0, _OW - 1.0)
    y1 = jnp.clip((cy + 0.5 * h) * _OH, 0.0, _OH - 1.0)
    boxes_out[...] = jnp.stack([x0, y0, x1, y1], axis=1)
    area_out[...] = (x1 - x0) * (y1 - y0)


def _split_hi_lo(x):
    hi = x.astype(jnp.bfloat16)
    lo = (x - hi.astype(jnp.float32)).astype(jnp.bfloat16)
    return hi, lo


def _mask_kernel(qref, sref, aref, mask_ref, wr_ref, wc_ref,
                 masks_out, score_out, keep_out):
    i = pl.program_id(0)
    m = mask_ref[0, 0]
    pos = (m > 0.0).astype(jnp.float32)
    ratio = jnp.sum(jax.nn.sigmoid(m) * pos) / (jnp.sum(pos) + 1e-6)
    mhi, mlo = _split_hi_lo(m)
    wr = wr_ref[...]
    dn1 = (((1,), (0,)), ((), ()))
    tmp = (jax.lax.dot_general(wr, mhi, dn1,
                               preferred_element_type=jnp.float32)
           + jax.lax.dot_general(wr, mlo, dn1,
                                 preferred_element_type=jnp.float32))
    thi, tlo = _split_hi_lo(tmp)
    wc = wc_ref[...]
    dn2 = (((1,), (1,)), ((), ()))
    up = (jax.lax.dot_general(thi, wc, dn2,
                              preferred_element_type=jnp.float32)
          + jax.lax.dot_general(tlo, wc, dn2,
                                preferred_element_type=jnp.float32))
    bm = up > 0.0
    masks_out[0] = bm
    msum = jnp.sum(bm.astype(jnp.float32))
    sc = sref[i] * ratio
    kp = (msum > 5.0) & (aref[i] > 10.0) & (sc > 0.05)
    score_out[0, 0] = jnp.full((128,), sc, jnp.float32)
    keep_out[0, 0] = jnp.full((128,), kp, jnp.bool_)


@functools.partial(jax.jit, static_argnames=())
def kernel(class_queries_logits, masks_queries_logits, pred_boxes):
    scores_r, labels_r, qidx_r, boxes_cm, area_r = pl.pallas_call(
        _topk_box_kernel,
        out_shape=(
            jax.ShapeDtypeStruct((_B, 128), jnp.float32),
            jax.ShapeDtypeStruct((_B, 128), jnp.int32),
            jax.ShapeDtypeStruct((_B, 128), jnp.int32),
            jax.ShapeDtypeStruct((_B, 4, 128), jnp.float32),
            jax.ShapeDtypeStruct((_B, 128), jnp.float32),
        ),
    )(class_queries_logits, pred_boxes)

    q_flat = qidx_r[:, :_K].reshape(-1)
    s_flat = scores_r[:, :_K].reshape(-1)
    a_flat = area_r[:, :_K].reshape(-1)
    w_up = jnp.asarray(_W_UP).astype(jnp.bfloat16)

    grid_spec = pltpu.PrefetchScalarGridSpec(
        num_scalar_prefetch=3,
        grid=(_B * _K,),
        in_specs=[
            pl.BlockSpec((1, 1, _MR, _MR),
                         lambda i, q, s, a: (i // _K, q[i], 0, 0)),
            pl.BlockSpec((_OH, _MR), lambda i, q, s, a: (0, 0)),
            pl.BlockSpec((_OW, _MR), lambda i, q, s, a: (0, 0)),
        ],
        out_specs=[
            pl.BlockSpec((1, _OH, _OW), lambda i, q, s, a: (i, 0, 0)),
            pl.BlockSpec((1, 1, 128), lambda i, q, s, a: (i, 0, 0)),
            pl.BlockSpec((1, 1, 128), lambda i, q, s, a: (i, 0, 0)),
        ],
    )
    masks_o, score_o, keep_o = pl.pallas_call(
        _mask_kernel,
        grid_spec=grid_spec,
        out_shape=(
            jax.ShapeDtypeStruct((_B * _K, _OH, _OW), jnp.bool_),
            jax.ShapeDtypeStruct((_B * _K, 1, 128), jnp.float32),
            jax.ShapeDtypeStruct((_B * _K, 1, 128), jnp.bool_),
        ),
        compiler_params=pltpu.CompilerParams(
            dimension_semantics=("arbitrary",)),
    )(q_flat, s_flat, a_flat, masks_queries_logits, w_up, w_up)

    pred_scores = score_o[:, 0, 0].reshape(_B, _K)
    keep = keep_o[:, 0, 0].reshape(_B, _K)
    masks = masks_o.reshape(_B, _K, _OH, _OW)
    boxes = boxes_cm.transpose(0, 2, 1)[:, :_K, :]
    labels = labels_r[:, :_K]
    return (pred_scores, boxes, labels, masks, keep)


# speedup vs baseline: 1.9352x; 1.2653x over previous
import functools

import jax
import jax.numpy as jnp
import numpy as np
from jax.experimental import pallas as pl
from jax.experimental.pallas import tpu as pltpu

_B = 4
_Q = 300
_C = 80
_K = 100
_MR = 128
_OH = 512
_OW = 512


def _resize_weights(in_size: int, out_size: int) -> np.ndarray:
    scale = out_size / in_size
    sample_f = (np.arange(out_size) + 0.5) / scale - 0.5
    x = np.abs(sample_f[:, None] - np.arange(in_size)[None, :])
    w = np.maximum(0.0, 1.0 - x)
    w = w / w.sum(axis=1, keepdims=True)
    return w.astype(np.float32)


_W_UP = _resize_weights(_MR, _OH)


def _topk_box_kernel(cls_ref, box_ref, scores_out, labels_out, qidx_out,
                     boxes_out, area_out):
    x = jax.nn.sigmoid(cls_ref[...])
    flat_i = (jax.lax.broadcasted_iota(jnp.int32, (_B, _Q, _C), 1) * _C
              + jax.lax.broadcasted_iota(jnp.int32, (_B, _Q, _C), 2))
    lanes = jax.lax.broadcasted_iota(jnp.int32, (_B, 128), 1)

    def body(k, carry):
        x, s_row, l_row, q_row = carry
        m = jnp.max(x, axis=(1, 2))
        m3 = m[:, None, None]
        idx = jnp.min(jnp.where(x == m3, flat_i, jnp.int32(2**30)),
                      axis=(1, 2))
        x = jnp.where(flat_i == idx[:, None, None], -1.0, x)
        kmask = lanes == k
        s_row = jnp.where(kmask, m[:, None], s_row)
        l_row = jnp.where(kmask, (idx % _C)[:, None], l_row)
        q_row = jnp.where(kmask, (idx // _C)[:, None], q_row)
        return x, s_row, l_row, q_row

    init = (x,
            jnp.zeros((_B, 128), jnp.float32),
            jnp.zeros((_B, 128), jnp.int32),
            jnp.zeros((_B, 128), jnp.int32))
    _, s_row, l_row, q_row = jax.lax.fori_loop(0, _K, body, init)

    scores_out[...] = s_row
    labels_out[...] = l_row
    qidx_out[...] = q_row

    q_iota = jax.lax.broadcasted_iota(jnp.int32, (_B, _Q, 128), 1)
    sel = (q_iota == q_row[:, None, :]).astype(jnp.float32)
    coords = [jnp.sum(sel * box_ref[:, :, j][:, :, None], axis=1)
              for j in range(4)]
    cx, cy, w, h = coords
    x0 = jnp.clip((cx - 0.5 * w) * _OW, 0.0, _OW - 1.0)
    y0 = jnp.clip((cy - 0.5 * h) * _OH, 0.0, _OH - 1.0)
    x1 = jnp.clip((cx + 0.5 * w) * _OW, 0.0, _OW - 1.0)
    y1 = jnp.clip((cy + 0.5 * h) * _OH, 0.0, _OH - 1.0)
    boxes_out[...] = jnp.stack([x0, y0, x1, y1], axis=1)
    area_out[...] = (x1 - x0) * (y1 - y0)


_NM = 4


def _row_upsample4(m):
    h, w = m.shape
    mp = jnp.concatenate([m[:1], m[:-1]], axis=0)
    mn = jnp.concatenate([m[1:], m[-1:]], axis=0)
    return jnp.stack([
        0.375 * mp + 0.625 * m,
        0.125 * mp + 0.875 * m,
        0.875 * m + 0.125 * mn,
        0.625 * m + 0.375 * mn,
    ], axis=1).reshape(4 * h, w)


def _split_hi_lo(x):
    hi = x.astype(jnp.bfloat16)
    lo = (x - hi.astype(jnp.float32)).astype(jnp.bfloat16)
    return hi, lo


def _mask_kernel(qref, sref, aref, m0_ref, m1_ref, m2_ref, m3_ref, wc_ref,
                 masks_out, score_out, keep_out):
    i = pl.program_id(0)
    wc = wc_ref[...]
    dn2 = (((1,), (1,)), ((), ()))
    for j, mref in enumerate((m0_ref, m1_ref, m2_ref, m3_ref)):
        m = mref[0, 0]
        pos = (m > 0.0).astype(jnp.float32)
        ratio = jnp.sum(jax.nn.sigmoid(m) * pos) / (jnp.sum(pos) + 1e-6)
        rows = _row_upsample4(m)
        thi, tlo = _split_hi_lo(rows)
        up = (jax.lax.dot_general(thi, wc, dn2,
                                  preferred_element_type=jnp.float32)
              + jax.lax.dot_general(tlo, wc, dn2,
                                    preferred_element_type=jnp.float32))
        bm = up > 0.0
        masks_out[j] = bm
        msum = jnp.sum(bm.astype(jnp.float32))
        sc = sref[i * _NM + j] * ratio
        kp = (msum > 5.0) & (aref[i * _NM + j] > 10.0) & (sc > 0.05)
        score_out[j, 0] = jnp.full((128,), sc, jnp.float32)
        keep_out[j, 0] = jnp.full((128,), kp, jnp.bool_)


@functools.partial(jax.jit, static_argnames=())
def kernel(class_queries_logits, masks_queries_logits, pred_boxes):
    scores_r, labels_r, qidx_r, boxes_cm, area_r = pl.pallas_call(
        _topk_box_kernel,
        out_shape=(
            jax.ShapeDtypeStruct((_B, 128), jnp.float32),
            jax.ShapeDtypeStruct((_B, 128), jnp.int32),
            jax.ShapeDtypeStruct((_B, 128), jnp.int32),
            jax.ShapeDtypeStruct((_B, 4, 128), jnp.float32),
            jax.ShapeDtypeStruct((_B, 128), jnp.float32),
        ),
    )(class_queries_logits, pred_boxes)

    q_flat = qidx_r[:, :_K].reshape(-1)
    s_flat = scores_r[:, :_K].reshape(-1)
    a_flat = area_r[:, :_K].reshape(-1)

    w_up = jnp.asarray(_W_UP).astype(jnp.bfloat16)

    def _mk_spec(j):
        return pl.BlockSpec(
            (1, 1, _MR, _MR),
            lambda i, q, s, a: ((i * _NM + j) // _K, q[i * _NM + j], 0, 0))

    grid_spec = pltpu.PrefetchScalarGridSpec(
        num_scalar_prefetch=3,
        grid=(_B * _K // _NM,),
        in_specs=[
            _mk_spec(0), _mk_spec(1), _mk_spec(2), _mk_spec(3),
            pl.BlockSpec((_OW, _MR), lambda i, q, s, a: (0, 0)),
        ],
        out_specs=[
            pl.BlockSpec((_NM, _OH, _OW), lambda i, q, s, a: (i, 0, 0)),
            pl.BlockSpec((_NM, 1, 128), lambda i, q, s, a: (i, 0, 0)),
            pl.BlockSpec((_NM, 1, 128), lambda i, q, s, a: (i, 0, 0)),
        ],
    )
    masks_o, score_o, keep_o = pl.pallas_call(
        _mask_kernel,
        grid_spec=grid_spec,
        out_shape=(
            jax.ShapeDtypeStruct((_B * _K, _OH, _OW), jnp.bool_),
            jax.ShapeDtypeStruct((_B * _K, 1, 128), jnp.float32),
            jax.ShapeDtypeStruct((_B * _K, 1, 128), jnp.bool_),
        ),
        compiler_params=pltpu.CompilerParams(
            dimension_semantics=("arbitrary",)),
    )(q_flat, s_flat, a_flat, masks_queries_logits,
      masks_queries_logits, masks_queries_logits, masks_queries_logits,
      w_up)

    pred_scores = score_o[:, 0, 0].reshape(_B, _K)
    keep = keep_o[:, 0, 0].reshape(_B, _K)
    masks = masks_o.reshape(_B, _K, _OH, _OW)
    boxes = boxes_cm.transpose(0, 2, 1)[:, :_K, :]
    labels = labels_r[:, :_K]
    return (pred_scores, boxes, labels, masks, keep)
